# baseline (speedup 1.0000x reference)
import functools

import jax
import jax.numpy as jnp
from jax import lax
from jax.experimental import pallas as pl
from jax.experimental.pallas import tpu as pltpu
from jax.experimental.pallas import tpu_sc as plsc

_B = 12
_WB = 128
_ZR = 128


def _sc_zero_mid(R, H, W, dtype):
    n_workers = 32
    rows_per_w = R // n_workers
    n_chunks = H // _ZR
    mesh = plsc.VectorSubcoreMesh(core_axis_name="c", subcore_axis_name="s")

    @functools.partial(
        pl.kernel, mesh=mesh,
        out_type=[
            jax.ShapeDtypeStruct((R, H, W), dtype),
            jax.ShapeDtypeStruct((R, H, W), dtype),
        ],
        scratch_types=[
            pltpu.VMEM((_ZR, _WB), dtype),
            pltpu.SemaphoreType.DMA,
        ],
    )
    def zero_mid(zl_hbm, zg_hbm, zbuf, sem):
        z16 = jnp.zeros((16,), dtype)
        for r in range(_ZR):
            for c in range(_WB // 16):
                zbuf[r, pl.ds(c * 16, 16)] = z16
        wid = lax.axis_index("s") * 2 + lax.axis_index("c")
        copies = []
        for k in range(rows_per_w):
            rr = wid * rows_per_w + k
            for q in range(n_chunks):
                for dst in (zl_hbm, zg_hbm):
                    c = pltpu.make_async_copy(
                        zbuf,
                        dst.at[rr, pl.ds(q * _ZR, _ZR), _WB:2 * _WB],
                        sem)
                    c.start()
                    copies.append(c)
        for c in copies:
            c.wait()

    return zero_mid()


def _tc_body(mk_ref, ms_ref, lst_ref, gui_ref, zl_ref, zg_ref,
             ol_ref, og_ref):
    j = pl.program_id(1)
    m = jnp.where(j == 0, ms_ref[0], mk_ref[0])
    a = jnp.where(j == 0, gui_ref[...], lst_ref[...])
    b = jnp.where(j == 0, lst_ref[...], gui_ref[...])
    ol_ref[...] = a * m
    og_ref[...] = b * m


def kernel(lst, gui):
    N, C, H, W = lst.shape
    R = N * C
    lst3 = lst.reshape(R, H, W)
    gui3 = gui.reshape(R, H, W)

    spatial_mask = jax.random.randint(
        jax.random.key(42), (H,), 0, 2, dtype=jnp.int32)
    has0 = jnp.any(spatial_mask == 0)
    has1 = jnp.any(spatial_mask == 1)
    col = jnp.arange(W)
    m_keep = jnp.where(((col == W - 1) & has0) | ((col == W - 2) & has1),
                       1.0, 0.0).astype(lst.dtype)
    m_swap = jnp.where(((col == 0) & has0) | ((col == 1) & has1),
                       1.0, 0.0).astype(lst.dtype)
    mk_hi = m_keep[W - _WB:].reshape(1, 1, _WB)
    ms_lo = m_swap[:_WB].reshape(1, 1, _WB)

    zl, zg = _sc_zero_mid(R, H, W, lst.dtype)

    grid = (R // _B, 2)
    in_spec = pl.BlockSpec((_B, H, _WB), lambda i, j: (i, 0, 2 * j))
    out_spec = pl.BlockSpec((_B, H, _WB), lambda i, j: (i, 0, 2 * j))
    any_spec = pl.BlockSpec(memory_space=pltpu.MemorySpace.HBM)
    vec_spec = pl.BlockSpec((1, 1, _WB), lambda i, j: (0, 0, 0))
    out_lst, out_gui = pl.pallas_call(
        _tc_body,
        grid=grid,
        in_specs=[vec_spec, vec_spec, in_spec, in_spec, any_spec, any_spec],
        out_specs=[out_spec, out_spec],
        out_shape=[
            jax.ShapeDtypeStruct((R, H, W), lst.dtype),
            jax.ShapeDtypeStruct((R, H, W), gui.dtype),
        ],
        input_output_aliases={4: 0, 5: 1},
    )(mk_hi, ms_lo, lst3, gui3, zl, zg)
    return (out_lst.reshape(N, C, H, W), out_gui.reshape(N, C, H, W))


# speedup vs baseline: 1.1193x; 1.1193x over previous
import jax
import jax.numpy as jnp
from jax.experimental import pallas as pl


def _body(mk_ref, ms_ref, lst_lo_ref, lst_hi_ref, gui_lo_ref, gui_hi_ref,
          ol_ref, og_ref):
    WB = 128
    mk = mk_ref[0]
    ms = ms_ref[0]
    zeros_mid = jnp.zeros_like(ol_ref[:, :, WB:2 * WB])
    ol_ref[:, :, 0:WB] = gui_lo_ref[...] * ms
    ol_ref[:, :, WB:2 * WB] = zeros_mid
    ol_ref[:, :, 2 * WB:3 * WB] = lst_hi_ref[...] * mk
    og_ref[:, :, 0:WB] = lst_lo_ref[...] * ms
    og_ref[:, :, WB:2 * WB] = zeros_mid
    og_ref[:, :, 2 * WB:3 * WB] = gui_hi_ref[...] * mk


def kernel(lst, gui):
    N, C, H, W = lst.shape
    R = N * C
    lst3 = lst.reshape(R, H, W)
    gui3 = gui.reshape(R, H, W)

    spatial_mask = jax.random.randint(
        jax.random.key(42), (H,), 0, 2, dtype=jnp.int32)
    has0 = jnp.any(spatial_mask == 0)
    has1 = jnp.any(spatial_mask == 1)
    col = jnp.arange(W)
    m_keep = jnp.where(((col == W - 1) & has0) | ((col == W - 2) & has1),
                       1.0, 0.0).astype(lst.dtype)
    m_swap = jnp.where(((col == 0) & has0) | ((col == 1) & has1),
                       1.0, 0.0).astype(lst.dtype)
    WB = 128
    mk_hi = m_keep[W - WB:].reshape(1, 1, WB)
    ms_lo = m_swap[:WB].reshape(1, 1, WB)

    B = 12
    HB = 192
    grid = (R // B, H // HB)
    lo_spec = pl.BlockSpec((B, HB, WB), lambda i, h: (i, h, 0))
    hi_spec = pl.BlockSpec((B, HB, WB), lambda i, h: (i, h, W // WB - 1))
    out_spec = pl.BlockSpec((B, HB, W), lambda i, h: (i, h, 0))
    vec_spec = pl.BlockSpec((1, 1, WB), lambda i, h: (0, 0, 0))
    out_lst, out_gui = pl.pallas_call(
        _body,
        grid=grid,
        in_specs=[vec_spec, vec_spec, lo_spec, hi_spec, lo_spec, hi_spec],
        out_specs=[out_spec, out_spec],
        out_shape=[
            jax.ShapeDtypeStruct((R, H, W), lst.dtype),
            jax.ShapeDtypeStruct((R, H, W), gui.dtype),
        ],
    )(mk_hi, ms_lo, lst3, lst3, gui3, gui3)
    return (out_lst.reshape(N, C, H, W), out_gui.reshape(N, C, H, W))


# speedup vs baseline: 1.1357x; 1.0147x over previous
import jax
import jax.numpy as jnp
from jax.experimental import pallas as pl


def _body(mk_ref, ms_ref, lst_lo_ref, lst_hi_ref, gui_lo_ref, gui_hi_ref,
          ol_ref, og_ref):
    WB = 128
    mk = mk_ref[0]
    ms = ms_ref[0]
    zeros_mid = jnp.zeros_like(ol_ref[:, :, WB:2 * WB])
    ol_ref[:, :, 0:WB] = gui_lo_ref[...] * ms
    ol_ref[:, :, WB:2 * WB] = zeros_mid
    ol_ref[:, :, 2 * WB:3 * WB] = lst_hi_ref[...] * mk
    og_ref[:, :, 0:WB] = lst_lo_ref[...] * ms
    og_ref[:, :, WB:2 * WB] = zeros_mid
    og_ref[:, :, 2 * WB:3 * WB] = gui_hi_ref[...] * mk


def kernel(lst, gui):
    N, C, H, W = lst.shape
    R = N * C
    lst3 = lst.reshape(R, H, W)
    gui3 = gui.reshape(R, H, W)

    spatial_mask = jax.random.randint(
        jax.random.key(42), (H,), 0, 2, dtype=jnp.int32)
    has0 = jnp.any(spatial_mask == 0)
    has1 = jnp.any(spatial_mask == 1)
    col = jnp.arange(W)
    m_keep = jnp.where(((col == W - 1) & has0) | ((col == W - 2) & has1),
                       1.0, 0.0).astype(lst.dtype)
    m_swap = jnp.where(((col == 0) & has0) | ((col == 1) & has1),
                       1.0, 0.0).astype(lst.dtype)
    WB = 128
    mk_hi = m_keep[W - WB:].reshape(1, 1, WB)
    ms_lo = m_swap[:WB].reshape(1, 1, WB)

    for B in (12, 8, 6, 4, 3, 2, 1):
        if R % B == 0:
            break
    grid = (R // B,)
    lo_spec = pl.BlockSpec((B, H, WB), lambda i: (i, 0, 0))
    hi_spec = pl.BlockSpec((B, H, WB), lambda i: (i, 0, W // WB - 1))
    out_spec = pl.BlockSpec((B, H, W), lambda i: (i, 0, 0))
    vec_spec = pl.BlockSpec((1, 1, WB), lambda i: (0, 0, 0))
    out_lst, out_gui = pl.pallas_call(
        _body,
        grid=grid,
        in_specs=[vec_spec, vec_spec, lo_spec, hi_spec, lo_spec, hi_spec],
        out_specs=[out_spec, out_spec],
        out_shape=[
            jax.ShapeDtypeStruct((R, H, W), lst.dtype),
            jax.ShapeDtypeStruct((R, H, W), gui.dtype),
        ],
    )(mk_hi, ms_lo, lst3, lst3, gui3, gui3)
    return (out_lst.reshape(N, C, H, W), out_gui.reshape(N, C, H, W))
